# Initial kernel scaffold; baseline (speedup 1.0000x reference)
import jax
import jax.numpy as jnp
from jax.experimental import pallas as pl


def kernel(x, edge_index, WA, bA, WX, bX, W, bW, W1, b1, gamma, beta, W2, b2):
    raise NotImplementedError("write your pallas kernel here")


# speedup vs baseline: 8.5291x; 8.5291x over previous
import functools
import math

import jax
import jax.numpy as jnp
from jax import lax
from jax.experimental import pallas as pl
from jax.experimental.pallas import tpu as pltpu
from jax.experimental.pallas import tpu_sc as plsc

N = 10000
E = 320000
D = 128
H = 128
OUT = 128
EPS = 1e-5

NC = 2
NS = 16
HH = H // NC
EPW = E // NS
K = 80
NCHUNK = EPW // K
NPAIR = NCHUNK // 2
RPT = 624
RPT_LAST = N - (NS - 1) * RPT


def _spmm_sc(wa_halves, row3, col3, zeros_rpt):
  mesh = plsc.VectorSubcoreMesh(core_axis_name="c", subcore_axis_name="s")

  @functools.partial(
      pl.kernel,
      out_type=jax.ShapeDtypeStruct((NC, N, HH), jnp.float32),
      mesh=mesh,
      scratch_types=[
          pltpu.VMEM((NCHUNK, K), jnp.int32),
          pltpu.VMEM((NCHUNK, K), jnp.int32),
          pltpu.VMEM((K, HH), jnp.float32),
          pltpu.VMEM((K, HH), jnp.float32),
          pltpu.VMEM_SHARED((N, HH), jnp.float32),
          pltpu.SemaphoreType.DMA,
          pltpu.SemaphoreType.DMA,
      ],
      compiler_params=pltpu.CompilerParams(use_tc_tiling_on_sc=False),
  )
  def k(wa_hbm, row_hbm, col_hbm, z_hbm, out_hbm,
        colv, rowv, buf0, buf1, acc, g0, g1):
    cid = lax.axis_index("c")
    sid = lax.axis_index("s")
    table = wa_hbm.at[cid]

    pltpu.sync_copy(col_hbm.at[sid], colv)
    pltpu.sync_copy(row_hbm.at[sid], rowv)

    @pl.when(sid < NS - 1)
    def _():
      pltpu.sync_copy(z_hbm.at[pl.ds(0, RPT)], acc.at[pl.ds(sid * RPT, RPT)])

    @pl.when(sid == NS - 1)
    def _():
      pltpu.sync_copy(z_hbm, acc.at[pl.ds(sid * RPT, RPT_LAST)])

    pltpu.async_copy(table.at[colv.at[0]], buf0, g0)
    plsc.subcore_barrier()

    def pair(jj, _):
      c0 = 2 * jj
      pltpu.async_copy(table.at[colv.at[c0 + 1]], buf1, g1)
      pltpu.make_async_copy(table.at[colv.at[c0]], buf0, g0).wait()
      pltpu.sync_copy(buf0, acc.at[rowv.at[c0]], add=True)

      @pl.when(c0 + 2 < NCHUNK)
      def _():
        pltpu.async_copy(table.at[colv.at[c0 + 2]], buf0, g0)

      pltpu.make_async_copy(table.at[colv.at[c0 + 1]], buf1, g1).wait()
      pltpu.sync_copy(buf1, acc.at[rowv.at[c0 + 1]], add=True)
      return 0

    lax.fori_loop(0, NPAIR, pair, 0)

    plsc.subcore_barrier()

    @pl.when(sid < NS - 1)
    def _():
      pltpu.sync_copy(acc.at[pl.ds(sid * RPT, RPT)],
                      out_hbm.at[cid].at[pl.ds(sid * RPT, RPT)])

    @pl.when(sid == NS - 1)
    def _():
      pltpu.sync_copy(acc.at[pl.ds(sid * RPT, RPT_LAST)],
                      out_hbm.at[cid].at[pl.ds(sid * RPT, RPT_LAST)])

  return k(wa_halves, row3, col3, zeros_rpt)


BM = 1000


def _dense_tc(parts, x, wxt, wat, wbt, w1t, w2t,
              bA2, bX2, bW2, b12, b22, g2, be2):
  def body(parts_ref, x_ref, wxt_ref, wat_ref, wbt_ref, w1t_ref, w2t_ref,
           bA_ref, bX_ref, bW_ref, b1_ref, b2_ref, g_ref, be_ref, o_ref):
    f32 = jnp.float32
    xA = jnp.concatenate([parts_ref[0], parts_ref[1]], axis=-1) + bA_ref[0]
    xX = jnp.dot(x_ref[...], wxt_ref[...], preferred_element_type=f32)
    xX = xX + bX_ref[0]
    h = jnp.dot(xA, wat_ref[...], preferred_element_type=f32)
    h = h + jnp.dot(xX, wbt_ref[...], preferred_element_type=f32)
    h = jax.nn.relu(h + bW_ref[0] + xA + xX)
    h1 = jax.nn.relu(jnp.dot(h, w1t_ref[...], preferred_element_type=f32)
                     + b1_ref[0])
    h1 = g_ref[0] * h1 * (1.0 / math.sqrt(1.0 + EPS)) + be_ref[0]
    o_ref[...] = (jnp.dot(h1, w2t_ref[...], preferred_element_type=f32)
                  + b2_ref[0])

  wspec = pl.BlockSpec((H, H), lambda i: (0, 0))
  vspec = pl.BlockSpec((1, H), lambda i: (0, 0))
  return pl.pallas_call(
      body,
      grid=(N // BM,),
      in_specs=[
          pl.BlockSpec((NC, BM, HH), lambda i: (0, i, 0)),
          pl.BlockSpec((BM, D), lambda i: (i, 0)),
          wspec, wspec, wspec, wspec, wspec,
          vspec, vspec, vspec, vspec, vspec, vspec, vspec,
      ],
      out_specs=pl.BlockSpec((BM, OUT), lambda i: (i, 0)),
      out_shape=jax.ShapeDtypeStruct((N, OUT), jnp.float32),
  )(parts, x, wxt, wat, wbt, w1t, w2t, bA2, bX2, bW2, b12, b22, g2, be2)


def kernel(x, edge_index, WA, bA, WX, bX, W, bW, W1, b1, gamma, beta, W2, b2):
  row = edge_index[0].astype(jnp.int32)
  col = edge_index[1].astype(jnp.int32)
  row = row - jnp.min(row)
  row3 = row.reshape(NS, NCHUNK, K)
  col3 = col.reshape(NS, NCHUNK, K)
  wa_halves = WA.reshape(NC, HH, N).transpose(0, 2, 1)
  zeros_rpt = jnp.zeros((RPT_LAST, HH), jnp.float32)

  parts = _spmm_sc(wa_halves, row3, col3, zeros_rpt)

  wt = W.T
  return _dense_tc(
      parts, x, WX.T, wt[:H], wt[H:], W1.T, W2.T,
      bA.reshape(1, H), bX.reshape(1, H), bW.reshape(1, H),
      b1.reshape(1, H), b2.reshape(1, OUT),
      gamma.reshape(1, H), beta.reshape(1, H))
